# baseline (speedup 1.0000x reference)
import jax
import jax.numpy as jnp
from jax.experimental import pallas as pl
from jax.experimental.pallas import tpu as pltpu

_CN = 2000
_NBUF = 8


def _gat_reduce_kernel(a_ref, ft_hbm, o_ref, buf, sem, *, deg, nchunks):
    j = pl.program_id(0)
    k = pl.program_id(1)
    s = j * deg + k
    total = nchunks * deg

    def start_copy(step, slot):
        k2 = step % deg
        j2 = step // deg
        pltpu.make_async_copy(
            ft_hbm.at[k2, pl.ds(j2 * _CN, _CN), :],
            buf.at[slot],
            sem.at[slot],
        ).start()

    @pl.when(s == 0)
    def _warmup():
        for t in range(min(_NBUF, total)):
            start_copy(t, t)

    slot = s % _NBUF
    pltpu.make_async_copy(
        ft_hbm.at[k, pl.ds(j * _CN, _CN), :], buf.at[slot], sem.at[slot]
    ).wait()

    @pl.when(k == 0)
    def _init():
        o_ref[...] = buf[slot]

    @pl.when(k != 0)
    def _acc():
        o_ref[...] += buf[slot]

    @pl.when(s + _NBUF < total)
    def _refill():
        start_copy(s + _NBUF, slot)

    @pl.when(k == deg - 1)
    def _finish():
        ablk = a_ref[...]
        e = jax.nn.softmax(ablk[None, :, :], axis=0)[0]
        o_ref[...] *= e


def kernel(a, ft):
    deg, n, d = ft.shape
    nchunks = n // _CN
    import functools
    body = functools.partial(_gat_reduce_kernel, deg=deg, nchunks=nchunks)
    return pl.pallas_call(
        body,
        grid=(nchunks, deg),
        in_specs=[
            pl.BlockSpec((_CN, 1), lambda j, k: (j, 0)),
            pl.BlockSpec(memory_space=pl.ANY),
        ],
        out_specs=pl.BlockSpec((_CN, d), lambda j, k: (j, 0)),
        out_shape=jax.ShapeDtypeStruct((n, d), ft.dtype),
        scratch_shapes=[
            pltpu.VMEM((_NBUF, _CN, d), jnp.float32),
            pltpu.SemaphoreType.DMA((_NBUF,)),
        ],
        compiler_params=pltpu.CompilerParams(
            dimension_semantics=("arbitrary", "arbitrary"),
        ),
    )(a, ft)
